# baseline (speedup 1.0000x reference)
import functools

import jax
import jax.numpy as jnp
from jax import lax
from jax.experimental import pallas as pl
from jax.experimental.pallas import tpu as pltpu
from jax.experimental.pallas import tpu_sc as plsc

N_EMB = 1000000
D = 64
B = 16384

NC, NS, L = 2, 16, 16
NW = NC * NS
BLK = 128
BPWK = 244
WB = 4
WIN = WB * BLK
NWIN = BPWK // WB
TAIL0 = NW * BPWK * BLK
TAILN = N_EMB - TAIL0
MCAP = 4096
NBKT = 64
BCAP = 64


def _make_gather_kernel():
    mesh = plsc.VectorSubcoreMesh(
        core_axis_name="c", subcore_axis_name="s",
        num_cores=NC, num_subcores=NS)

    @functools.partial(
        pl.kernel,
        out_type=(jax.ShapeDtypeStruct((B + 8, D), jnp.float32),
                  jax.ShapeDtypeStruct((B + 8, D), jnp.float32)),
        mesh=mesh,
        scratch_types=[
            pltpu.VMEM((B,), jnp.int32),
            pltpu.VMEM((B,), jnp.int32),
            pltpu.VMEM((MCAP,), jnp.int32),
            pltpu.VMEM((MCAP,), jnp.int32),
            pltpu.VMEM((MCAP,), jnp.int32),
            pltpu.VMEM((MCAP,), jnp.int32),
            pltpu.VMEM((NBKT * BCAP + L,), jnp.int32),
            pltpu.VMEM((NBKT * BCAP + L,), jnp.int32),
            pltpu.SMEM((NBKT,), jnp.int32),
            pltpu.SMEM((NBKT,), jnp.int32),
            pltpu.VMEM((D, TAILN), jnp.float32),
            pltpu.VMEM((L, D), jnp.float32),
            pltpu.SemaphoreType.DMA,
        ],
        compiler_params=pltpu.CompilerParams(
            needs_layout_passes=False, use_tc_tiling_on_sc=True),
    )
    def gather_kernel(embt, sid, tid, souts, touts, sidv, tidv,
                      msid, mspid, mtid, mtpid, sbkt, tbkt, scnt, tcnt,
                      win, stag, sem):
        w = lax.axis_index("s") * NC + lax.axis_index("c")
        lo_blk = w * BPWK
        base_id = lo_blk * BLK
        is_last = w == NW - 1
        hi_blk = lo_blk + BPWK + jnp.where(is_last, 5, 0)

        pltpu.sync_copy(sid, sidv)
        pltpu.sync_copy(tid, tidv)
        for b in range(NBKT):
            scnt[b] = 0
            tcnt[b] = 0

        lanes = lax.iota(jnp.int32, L)

        def scan_body(i, offs):
            off_s, off_t = offs
            pidx = lanes + i * L
            s16 = sidv[pl.ds(i * L, L)]
            t16 = tidv[pl.ds(i * L, L)]
            sblk = lax.shift_right_logical(s16, 7)
            tblk = lax.shift_right_logical(t16, 7)
            ms = (sblk >= lo_blk) & (sblk < hi_blk)
            mt = (tblk >= lo_blk) & (tblk < hi_blk)
            plsc.store_compressed(msid.at[pl.ds(off_s, L)], s16, mask=ms)
            plsc.store_compressed(mspid.at[pl.ds(off_s, L)], pidx, mask=ms)
            plsc.store_compressed(mtid.at[pl.ds(off_t, L)], t16, mask=mt)
            plsc.store_compressed(mtpid.at[pl.ds(off_t, L)], pidx, mask=mt)
            ns = plsc.all_reduce_population_count(ms)[0]
            nt = plsc.all_reduce_population_count(mt)[0]
            off_s = jnp.minimum(off_s + ns, MCAP - L)
            off_t = jnp.minimum(off_t + nt, MCAP - L)
            return (off_s, off_t)

        cnt_s, cnt_t = lax.fori_loop(
            0, B // L, scan_body, (jnp.int32(0), jnp.int32(0)))

        def bucketize(mid, mpid, cnt, bkt, bcnt):
            def bk_body(k, carry):
                ids16 = mid[pl.ds(k * L, L)]
                pid16 = mpid[pl.ds(k * L, L)]
                valid = (lanes + k * L) < cnt
                lid16 = ids16 - base_id
                win16 = lax.shift_right_logical(lid16, 9)
                win16 = jnp.where(valid, win16, NBKT - 1)
                packed = (lid16 << 14) | pid16
                for j in range(L):
                    wv = win16[j]
                    slot = bcnt[wv]
                    bcnt[wv] = slot + 1
                    pos = wv * BCAP + jnp.minimum(slot, BCAP - 1)
                    plsc.store_compressed(
                        bkt.at[pl.ds(pos, L)], packed, mask=lanes == j)
                return carry

            lax.fori_loop(0, (cnt + L - 1) // L, bk_body, 0)

        bucketize(msid, mspid, cnt_s, sbkt, scnt)
        bucketize(mtid, mtpid, cnt_t, tbkt, tcnt)

        def drain(bkt, n, wbase, width, out_hbm):
            def chunk(k, carry):
                packed = bkt[pl.ds(wbase * BCAP + k * L, L)]
                lid16 = lax.shift_right_logical(packed, 14)
                pid16 = packed & (B - 1)
                valid = (lanes + k * L) < n
                col16 = jnp.clip(lid16 - wbase * WIN, 0, width - 1)
                pid16 = jnp.where(valid, pid16, B)
                copies = []
                for j in range(L):
                    cols = jnp.full((L,), col16[j], jnp.int32)
                    for q in range(D // L):
                        v = plsc.load_gather(win, [lanes + q * L, cols])
                        stag[j, pl.ds(q * L, L)] = v
                    copies.append(pltpu.async_copy(
                        stag.at[j], out_hbm.at[pid16[j]], sem))
                for c in copies:
                    c.wait()
                return carry

            lax.fori_loop(0, (n + L - 1) // L, chunk, 0)

        def window_body(i, carry):
            b0 = pl.multiple_of(base_id + i * WIN, BLK)
            pltpu.sync_copy(embt.at[:, pl.ds(b0, WIN)],
                            win.at[:, pl.ds(0, WIN)])
            drain(sbkt, jnp.minimum(scnt[i], BCAP), i, WIN, souts)
            drain(tbkt, jnp.minimum(tcnt[i], BCAP), i, WIN, touts)
            return carry

        lax.fori_loop(0, NWIN, window_body, 0)

        @pl.when(is_last)
        def _tail():
            pltpu.sync_copy(embt.at[:, pl.ds(TAIL0, TAILN - BLK // 2)],
                            win.at[:, pl.ds(0, TAILN - BLK // 2)])
            pltpu.sync_copy(
                embt.at[:, pl.ds(TAIL0 + TAILN - BLK // 2, BLK // 2)],
                win.at[:, pl.ds(TAILN - BLK // 2, BLK // 2)])

            def tail_drain(bkt, bcnt, out_hbm):
                for tw in (NWIN, NWIN + 1):
                    n = jnp.minimum(bcnt[tw], BCAP)

                    def chunk(k, carry):
                        packed = bkt[pl.ds(tw * BCAP + k * L, L)]
                        lid16 = lax.shift_right_logical(packed, 14)
                        pid16 = packed & (B - 1)
                        valid = (lanes + k * L) < n
                        col16 = jnp.clip(lid16 - NWIN * WIN, 0, TAILN - 1)
                        pid16 = jnp.where(valid, pid16, B)
                        copies = []
                        for j in range(L):
                            cols = jnp.full((L,), col16[j], jnp.int32)
                            for q in range(D // L):
                                v = plsc.load_gather(
                                    win, [lanes + q * L, cols])
                                stag[j, pl.ds(q * L, L)] = v
                            copies.append(pltpu.async_copy(
                                stag.at[j], out_hbm.at[pid16[j]], sem))
                        for c in copies:
                            c.wait()
                        return carry

                    lax.fori_loop(0, (n + L - 1) // L, chunk, 0)

            tail_drain(sbkt, scnt, souts)
            tail_drain(tbkt, tcnt, touts)

    return gather_kernel


_gather_call = _make_gather_kernel()


def _loss_body(s_ref, t_ref, td_ref, cf_ref, out_ref):
    d = s_ref[pl.ds(0, B), :] - t_ref[pl.ds(0, B), :]
    ssq = jnp.sum(d * d, axis=1)
    err = jnp.sqrt(ssq) * 0.125 - td_ref[...]
    out_ref[0, 0] = jnp.sum(err * err * cf_ref[...]) * (1.0 / B)


_loss_call = pl.pallas_call(
    _loss_body,
    out_shape=jax.ShapeDtypeStruct((1, 1), jnp.float32),
    out_specs=pl.BlockSpec(memory_space=pltpu.SMEM),
)


def kernel(embeddings, source_id, target_id, target_distance, confidence):
    sid = source_id.astype(jnp.int32)
    tid = target_id.astype(jnp.int32)
    srows, trows = _gather_call(embeddings.T, sid, tid)
    loss = _loss_call(srows, trows, target_distance, confidence)
    return loss[0, 0]


# speedup vs baseline: 1.0009x; 1.0009x over previous
import functools

import jax
import jax.numpy as jnp
from jax import lax
from jax.experimental import pallas as pl
from jax.experimental.pallas import tpu as pltpu
from jax.experimental.pallas import tpu_sc as plsc

N_EMB = 1000000
D = 64
B = 16384

NC, NS, L = 2, 16, 16
NW = NC * NS
BLK = 128
BPWK = 244
WB = 4
WIN = WB * BLK
NWIN = BPWK // WB
TAIL0 = NW * BPWK * BLK
TAILN = N_EMB - TAIL0
MCAP = 4096
NBKT = 64
BCAP = 64


def _make_gather_kernel():
    mesh = plsc.VectorSubcoreMesh(
        core_axis_name="c", subcore_axis_name="s",
        num_cores=NC, num_subcores=NS)

    @functools.partial(
        pl.kernel,
        out_type=(jax.ShapeDtypeStruct((B + 8, D), jnp.float32),
                  jax.ShapeDtypeStruct((B + 8, D), jnp.float32)),
        mesh=mesh,
        scratch_types=[
            pltpu.VMEM((B,), jnp.int32),
            pltpu.VMEM((B,), jnp.int32),
            pltpu.VMEM((MCAP,), jnp.int32),
            pltpu.VMEM((MCAP,), jnp.int32),
            pltpu.VMEM((MCAP,), jnp.int32),
            pltpu.VMEM((MCAP,), jnp.int32),
            pltpu.VMEM((NBKT * BCAP + L,), jnp.int32),
            pltpu.VMEM((NBKT * BCAP + L,), jnp.int32),
            pltpu.SMEM((NBKT,), jnp.int32),
            pltpu.SMEM((NBKT,), jnp.int32),
            pltpu.VMEM((D, TAILN), jnp.float32),
            pltpu.VMEM((L, D), jnp.float32),
            pltpu.SemaphoreType.DMA,
        ],
        compiler_params=pltpu.CompilerParams(
            needs_layout_passes=False, use_tc_tiling_on_sc=True),
    )
    def gather_kernel(embt, sid, tid, souts, touts, sidv, tidv,
                      msid, mspid, mtid, mtpid, sbkt, tbkt, scnt, tcnt,
                      win, stag, sem):
        w = lax.axis_index("s") * NC + lax.axis_index("c")
        lo_blk = w * BPWK
        base_id = lo_blk * BLK
        is_last = w == NW - 1
        hi_blk = lo_blk + BPWK + jnp.where(is_last, 5, 0)

        pltpu.sync_copy(sid, sidv)
        pltpu.sync_copy(tid, tidv)
        for b in range(NBKT):
            scnt[b] = 0
            tcnt[b] = 0

        lanes = lax.iota(jnp.int32, L)

        def scan_body(i, offs):
            off_s, off_t = offs
            pidx = lanes + i * L
            s16 = sidv[pl.ds(i * L, L)]
            t16 = tidv[pl.ds(i * L, L)]
            sblk = lax.shift_right_logical(s16, 7)
            tblk = lax.shift_right_logical(t16, 7)
            ms = (sblk >= lo_blk) & (sblk < hi_blk)
            mt = (tblk >= lo_blk) & (tblk < hi_blk)
            plsc.store_compressed(msid.at[pl.ds(off_s, L)], s16, mask=ms)
            plsc.store_compressed(mspid.at[pl.ds(off_s, L)], pidx, mask=ms)
            plsc.store_compressed(mtid.at[pl.ds(off_t, L)], t16, mask=mt)
            plsc.store_compressed(mtpid.at[pl.ds(off_t, L)], pidx, mask=mt)
            ns = plsc.all_reduce_population_count(ms)[0]
            nt = plsc.all_reduce_population_count(mt)[0]
            off_s = jnp.minimum(off_s + ns, MCAP - L)
            off_t = jnp.minimum(off_t + nt, MCAP - L)
            return (off_s, off_t)

        with jax.named_scope("ph_scan"):
            cnt_s, cnt_t = lax.fori_loop(
                0, B // L, scan_body, (jnp.int32(0), jnp.int32(0)))

        def bucketize(mid, mpid, cnt, bkt, bcnt):
            def bk_body(k, carry):
                ids16 = mid[pl.ds(k * L, L)]
                pid16 = mpid[pl.ds(k * L, L)]
                valid = (lanes + k * L) < cnt
                lid16 = ids16 - base_id
                win16 = lax.shift_right_logical(lid16, 9)
                win16 = jnp.where(valid, win16, NBKT - 1)
                packed = (lid16 << 14) | pid16
                for j in range(L):
                    wv = win16[j]
                    slot = bcnt[wv]
                    bcnt[wv] = slot + 1
                    pos = wv * BCAP + jnp.minimum(slot, BCAP - 1)
                    plsc.store_compressed(
                        bkt.at[pl.ds(pos, L)], packed, mask=lanes == j)
                return carry

            lax.fori_loop(0, (cnt + L - 1) // L, bk_body, 0)

        with jax.named_scope("ph_bucket"):
            bucketize(msid, mspid, cnt_s, sbkt, scnt)
            bucketize(mtid, mtpid, cnt_t, tbkt, tcnt)

        def drain(bkt, n, wbase, width, out_hbm):
            def chunk(k, carry):
                packed = bkt[pl.ds(wbase * BCAP + k * L, L)]
                lid16 = lax.shift_right_logical(packed, 14)
                pid16 = packed & (B - 1)
                valid = (lanes + k * L) < n
                col16 = jnp.clip(lid16 - wbase * WIN, 0, width - 1)
                pid16 = jnp.where(valid, pid16, B)
                copies = []
                for j in range(L):
                    cols = jnp.full((L,), col16[j], jnp.int32)
                    for q in range(D // L):
                        v = plsc.load_gather(win, [lanes + q * L, cols])
                        stag[j, pl.ds(q * L, L)] = v
                    copies.append(pltpu.async_copy(
                        stag.at[j], out_hbm.at[pid16[j]], sem))
                for c in copies:
                    c.wait()
                return carry

            lax.fori_loop(0, (n + L - 1) // L, chunk, 0)

        def window_body(i, carry):
            b0 = pl.multiple_of(base_id + i * WIN, BLK)
            with jax.named_scope("ph_windma"):
                pltpu.sync_copy(embt.at[:, pl.ds(b0, WIN)],
                                win.at[:, pl.ds(0, WIN)])
            with jax.named_scope("ph_drain"):
                drain(sbkt, jnp.minimum(scnt[i], BCAP), i, WIN, souts)
                drain(tbkt, jnp.minimum(tcnt[i], BCAP), i, WIN, touts)
            return carry

        with jax.named_scope("ph_windows"):
            lax.fori_loop(0, NWIN, window_body, 0)

        @pl.when(is_last)
        def _tail():
            pltpu.sync_copy(embt.at[:, pl.ds(TAIL0, TAILN - BLK // 2)],
                            win.at[:, pl.ds(0, TAILN - BLK // 2)])
            pltpu.sync_copy(
                embt.at[:, pl.ds(TAIL0 + TAILN - BLK // 2, BLK // 2)],
                win.at[:, pl.ds(TAILN - BLK // 2, BLK // 2)])

            def tail_drain(bkt, bcnt, out_hbm):
                for tw in (NWIN, NWIN + 1):
                    n = jnp.minimum(bcnt[tw], BCAP)

                    def chunk(k, carry):
                        packed = bkt[pl.ds(tw * BCAP + k * L, L)]
                        lid16 = lax.shift_right_logical(packed, 14)
                        pid16 = packed & (B - 1)
                        valid = (lanes + k * L) < n
                        col16 = jnp.clip(lid16 - NWIN * WIN, 0, TAILN - 1)
                        pid16 = jnp.where(valid, pid16, B)
                        copies = []
                        for j in range(L):
                            cols = jnp.full((L,), col16[j], jnp.int32)
                            for q in range(D // L):
                                v = plsc.load_gather(
                                    win, [lanes + q * L, cols])
                                stag[j, pl.ds(q * L, L)] = v
                            copies.append(pltpu.async_copy(
                                stag.at[j], out_hbm.at[pid16[j]], sem))
                        for c in copies:
                            c.wait()
                        return carry

                    lax.fori_loop(0, (n + L - 1) // L, chunk, 0)

            tail_drain(sbkt, scnt, souts)
            tail_drain(tbkt, tcnt, touts)

    return gather_kernel


_gather_call = _make_gather_kernel()


def _loss_body(s_ref, t_ref, td_ref, cf_ref, out_ref):
    d = s_ref[pl.ds(0, B), :] - t_ref[pl.ds(0, B), :]
    ssq = jnp.sum(d * d, axis=1)
    err = jnp.sqrt(ssq) * 0.125 - td_ref[...]
    out_ref[0, 0] = jnp.sum(err * err * cf_ref[...]) * (1.0 / B)


_loss_call = pl.pallas_call(
    _loss_body,
    out_shape=jax.ShapeDtypeStruct((1, 1), jnp.float32),
    out_specs=pl.BlockSpec(memory_space=pltpu.SMEM),
)


def kernel(embeddings, source_id, target_id, target_distance, confidence):
    sid = source_id.astype(jnp.int32)
    tid = target_id.astype(jnp.int32)
    srows, trows = _gather_call(embeddings.T, sid, tid)
    loss = _loss_call(srows, trows, target_distance, confidence)
    return loss[0, 0]


# speedup vs baseline: 1.0013x; 1.0004x over previous
import functools

import jax
import jax.numpy as jnp
from jax import lax
from jax.experimental import pallas as pl
from jax.experimental.pallas import tpu as pltpu
from jax.experimental.pallas import tpu_sc as plsc

N_EMB = 1000000
D = 64
B = 16384

NC, NS, L = 2, 16, 16
NW = NC * NS
BLK = 128
BPWK = 244
WB = 4
WIN = WB * BLK
NWIN = BPWK // WB
TAIL0 = NW * BPWK * BLK
TAILN = N_EMB - TAIL0
MCAP = 4096
NBKT = 64
BCAP = 64


def _make_gather_kernel():
    mesh = plsc.VectorSubcoreMesh(
        core_axis_name="c", subcore_axis_name="s",
        num_cores=NC, num_subcores=NS)

    @functools.partial(
        pl.kernel,
        out_type=(jax.ShapeDtypeStruct((B + 8, D), jnp.float32),
                  jax.ShapeDtypeStruct((B + 8, D), jnp.float32)),
        mesh=mesh,
        scratch_types=[
            pltpu.VMEM((B,), jnp.int32),
            pltpu.VMEM((B,), jnp.int32),
            pltpu.VMEM((MCAP,), jnp.int32),
            pltpu.VMEM((MCAP,), jnp.int32),
            pltpu.VMEM((MCAP,), jnp.int32),
            pltpu.VMEM((MCAP,), jnp.int32),
            pltpu.VMEM((NBKT * BCAP + L,), jnp.int32),
            pltpu.VMEM((NBKT * BCAP + L,), jnp.int32),
            pltpu.SMEM((NBKT,), jnp.int32),
            pltpu.SMEM((NBKT,), jnp.int32),
            pltpu.VMEM((D, TAILN), jnp.float32),
            pltpu.VMEM((8 * L, D), jnp.float32),
            pltpu.SemaphoreType.DMA,
        ],
        compiler_params=pltpu.CompilerParams(
            needs_layout_passes=False, use_tc_tiling_on_sc=True),
    )
    def gather_kernel(embt, sid, tid, souts, touts, sidv, tidv,
                      msid, mspid, mtid, mtpid, sbkt, tbkt, scnt, tcnt,
                      win, stag, sem):
        w = lax.axis_index("s") * NC + lax.axis_index("c")
        lo_blk = w * BPWK
        base_id = lo_blk * BLK
        is_last = w == NW - 1
        hi_blk = lo_blk + BPWK + jnp.where(is_last, 5, 0)

        pltpu.sync_copy(sid, sidv)
        pltpu.sync_copy(tid, tidv)
        for b in range(NBKT):
            scnt[b] = 0
            tcnt[b] = 0

        lanes = lax.iota(jnp.int32, L)

        def scan_body(i, offs):
            off_s, off_t = offs
            pidx = lanes + i * L
            s16 = sidv[pl.ds(i * L, L)]
            t16 = tidv[pl.ds(i * L, L)]
            sblk = lax.shift_right_logical(s16, 7)
            tblk = lax.shift_right_logical(t16, 7)
            ms = (sblk >= lo_blk) & (sblk < hi_blk)
            mt = (tblk >= lo_blk) & (tblk < hi_blk)
            plsc.store_compressed(msid.at[pl.ds(off_s, L)], s16, mask=ms)
            plsc.store_compressed(mspid.at[pl.ds(off_s, L)], pidx, mask=ms)
            plsc.store_compressed(mtid.at[pl.ds(off_t, L)], t16, mask=mt)
            plsc.store_compressed(mtpid.at[pl.ds(off_t, L)], pidx, mask=mt)
            ns = plsc.all_reduce_population_count(ms)[0]
            nt = plsc.all_reduce_population_count(mt)[0]
            off_s = jnp.minimum(off_s + ns, MCAP - L)
            off_t = jnp.minimum(off_t + nt, MCAP - L)
            return (off_s, off_t)

        with jax.named_scope("ph_scan"):
            cnt_s, cnt_t = lax.fori_loop(
                0, B // L, scan_body, (jnp.int32(0), jnp.int32(0)))

        def bucketize(mid, mpid, cnt, bkt, bcnt):
            def bk_body(k, carry):
                ids16 = mid[pl.ds(k * L, L)]
                pid16 = mpid[pl.ds(k * L, L)]
                valid = (lanes + k * L) < cnt
                lid16 = ids16 - base_id
                win16 = lax.shift_right_logical(lid16, 9)
                win16 = jnp.where(valid, win16, NBKT - 1)
                packed = (lid16 << 14) | pid16
                for j in range(L):
                    wv = win16[j]
                    slot = bcnt[wv]
                    bcnt[wv] = slot + 1
                    pos = wv * BCAP + jnp.minimum(slot, BCAP - 1)
                    plsc.store_compressed(
                        bkt.at[pl.ds(pos, L)], packed, mask=lanes == j)
                return carry

            lax.fori_loop(0, (cnt + L - 1) // L, bk_body, 0)

        with jax.named_scope("ph_bucket"):
            bucketize(msid, mspid, cnt_s, sbkt, scnt)
            bucketize(mtid, mtpid, cnt_t, tbkt, tcnt)

        def ring_wait(_, c):
            pltpu.make_async_copy(souts.at[B], stag.at[0], sem).wait()
            return c

        def drain(bkt, n, bidx, cb, width, out_hbm, gk):
            def chunk(k, gk):
                packed = bkt[pl.ds(bidx * BCAP + k * L, L)]
                lid16 = lax.shift_right_logical(packed, 14)
                pid16 = packed & (B - 1)
                valid = (lanes + k * L) < n
                col16 = jnp.clip(lid16 - cb, 0, width - 1)
                pid16 = jnp.where(valid, pid16, B)
                lax.fori_loop(0, jnp.where(gk >= 8, L, 0), ring_wait, 0)
                ring = (gk & 7) * L
                for j in range(L):
                    cols = jnp.full((L,), col16[j], jnp.int32)
                    for q in range(D // L):
                        v = plsc.load_gather(win, [lanes + q * L, cols])
                        stag[ring + j, pl.ds(q * L, L)] = v
                    pltpu.async_copy(
                        stag.at[ring + j], out_hbm.at[pid16[j]], sem)
                return gk + 1

            return lax.fori_loop(0, (n + L - 1) // L, chunk, gk)

        def window_body(i, gk):
            b0 = pl.multiple_of(base_id + i * WIN, BLK)
            pltpu.sync_copy(embt.at[:, pl.ds(b0, WIN)],
                            win.at[:, pl.ds(0, WIN)])
            gk = drain(sbkt, jnp.minimum(scnt[i], BCAP), i, i * WIN,
                       WIN, souts, gk)
            gk = drain(tbkt, jnp.minimum(tcnt[i], BCAP), i, i * WIN,
                       WIN, touts, gk)
            return gk

        with jax.named_scope("ph_windows"):
            gk = lax.fori_loop(0, NWIN, window_body, jnp.int32(0))

        @pl.when(is_last)
        def _tail():
            pltpu.sync_copy(embt.at[:, pl.ds(TAIL0, TAILN - BLK // 2)],
                            win.at[:, pl.ds(0, TAILN - BLK // 2)])
            pltpu.sync_copy(
                embt.at[:, pl.ds(TAIL0 + TAILN - BLK // 2, BLK // 2)],
                win.at[:, pl.ds(TAILN - BLK // 2, BLK // 2)])
            g2 = gk
            for tw in (NWIN, NWIN + 1):
                g2 = drain(sbkt, jnp.minimum(scnt[tw], BCAP), tw,
                           NWIN * WIN, TAILN, souts, g2)
                g2 = drain(tbkt, jnp.minimum(tcnt[tw], BCAP), tw,
                           NWIN * WIN, TAILN, touts, g2)
            lax.fori_loop(0, L * jnp.minimum(g2, 8), ring_wait, 0)

        @pl.when(jnp.logical_not(is_last))
        def _final_drain():
            lax.fori_loop(0, L * jnp.minimum(gk, 8), ring_wait, 0)

    return gather_kernel


_gather_call = _make_gather_kernel()


def _loss_body(s_ref, t_ref, td_ref, cf_ref, out_ref):
    d = s_ref[pl.ds(0, B), :] - t_ref[pl.ds(0, B), :]
    ssq = jnp.sum(d * d, axis=1)
    err = jnp.sqrt(ssq) * 0.125 - td_ref[...]
    out_ref[0, 0] = jnp.sum(err * err * cf_ref[...]) * (1.0 / B)


_loss_call = pl.pallas_call(
    _loss_body,
    out_shape=jax.ShapeDtypeStruct((1, 1), jnp.float32),
    out_specs=pl.BlockSpec(memory_space=pltpu.SMEM),
)


def kernel(embeddings, source_id, target_id, target_distance, confidence):
    sid = source_id.astype(jnp.int32)
    tid = target_id.astype(jnp.int32)
    srows, trows = _gather_call(embeddings.T, sid, tid)
    loss = _loss_call(srows, trows, target_distance, confidence)
    return loss[0, 0]


# speedup vs baseline: 1.0191x; 1.0178x over previous
import functools

import jax
import jax.numpy as jnp
from jax import lax
from jax.experimental import pallas as pl
from jax.experimental.pallas import tpu as pltpu
from jax.experimental.pallas import tpu_sc as plsc

N_EMB = 1000000
D = 64
B = 16384

NC, NS, L = 2, 16, 16
NW = NC * NS
BLK = 128
BPWK = 244
WB = 4
WIN = WB * BLK
NWIN = BPWK // WB
TAIL0 = NW * BPWK * BLK
TAILN = N_EMB - TAIL0
MCAP = 4096
NBKT = 64
BCAP = 64


def _make_gather_kernel():
    mesh = plsc.VectorSubcoreMesh(
        core_axis_name="c", subcore_axis_name="s",
        num_cores=NC, num_subcores=NS)

    @functools.partial(
        pl.kernel,
        out_type=(jax.ShapeDtypeStruct((B + 8, 2 * D), jnp.float32),
                  jax.ShapeDtypeStruct((B + 8, 2 * D), jnp.float32)),
        mesh=mesh,
        scratch_types=[
            pltpu.VMEM((B,), jnp.int32),
            pltpu.VMEM((B,), jnp.int32),
            pltpu.VMEM((MCAP,), jnp.int32),
            pltpu.VMEM((MCAP,), jnp.int32),
            pltpu.VMEM((MCAP,), jnp.int32),
            pltpu.VMEM((MCAP,), jnp.int32),
            pltpu.VMEM((NBKT * BCAP + L,), jnp.int32),
            pltpu.VMEM((NBKT * BCAP + L,), jnp.int32),
            pltpu.SMEM((NBKT,), jnp.int32),
            pltpu.SMEM((NBKT,), jnp.int32),
            pltpu.VMEM((D, TAILN), jnp.float32),
            pltpu.VMEM((8 * L, 2 * D), jnp.float32),
            pltpu.VMEM((8, L), jnp.int32),
            pltpu.SemaphoreType.DMA,
        ],
        compiler_params=pltpu.CompilerParams(
            needs_layout_passes=False, use_tc_tiling_on_sc=True),
    )
    def gather_kernel(embt, sid, tid, souts, touts, sidv, tidv,
                      msid, mspid, mtid, mtpid, sbkt, tbkt, scnt, tcnt,
                      win, stag, pidr, sem):
        w = lax.axis_index("s") * NC + lax.axis_index("c")
        lo_blk = w * BPWK
        base_id = lo_blk * BLK
        is_last = w == NW - 1
        hi_blk = lo_blk + BPWK + jnp.where(is_last, 5, 0)

        pltpu.sync_copy(sid, sidv)
        pltpu.sync_copy(tid, tidv)
        for b in range(NBKT):
            scnt[b] = 0
            tcnt[b] = 0

        lanes = lax.iota(jnp.int32, L)

        def scan_body(i, offs):
            off_s, off_t = offs
            pidx = lanes + i * L
            s16 = sidv[pl.ds(i * L, L)]
            t16 = tidv[pl.ds(i * L, L)]
            sblk = lax.shift_right_logical(s16, 7)
            tblk = lax.shift_right_logical(t16, 7)
            ms = (sblk >= lo_blk) & (sblk < hi_blk)
            mt = (tblk >= lo_blk) & (tblk < hi_blk)
            plsc.store_compressed(msid.at[pl.ds(off_s, L)], s16, mask=ms)
            plsc.store_compressed(mspid.at[pl.ds(off_s, L)], pidx, mask=ms)
            plsc.store_compressed(mtid.at[pl.ds(off_t, L)], t16, mask=mt)
            plsc.store_compressed(mtpid.at[pl.ds(off_t, L)], pidx, mask=mt)
            ns = plsc.all_reduce_population_count(ms)[0]
            nt = plsc.all_reduce_population_count(mt)[0]
            off_s = jnp.minimum(off_s + ns, MCAP - L)
            off_t = jnp.minimum(off_t + nt, MCAP - L)
            return (off_s, off_t)

        with jax.named_scope("ph_scan"):
            cnt_s, cnt_t = lax.fori_loop(
                0, B // L, scan_body, (jnp.int32(0), jnp.int32(0)))

        def bucketize(mid, mpid, cnt, bkt, bcnt):
            def bk_body(k, carry):
                ids16 = mid[pl.ds(k * L, L)]
                pid16 = mpid[pl.ds(k * L, L)]
                valid = (lanes + k * L) < cnt
                lid16 = ids16 - base_id
                win16 = lax.shift_right_logical(lid16, 9)
                win16 = jnp.where(valid, win16, NBKT - 1)
                packed = (lid16 << 14) | pid16
                for j in range(L):
                    wv = win16[j]
                    slot = bcnt[wv]
                    bcnt[wv] = slot + 1
                    pos = wv * BCAP + jnp.minimum(slot, BCAP - 1)
                    plsc.store_compressed(
                        bkt.at[pl.ds(pos, L)], packed, mask=lanes == j)
                return carry

            lax.fori_loop(0, (cnt + L - 1) // L, bk_body, 0)

        with jax.named_scope("ph_bucket"):
            bucketize(msid, mspid, cnt_s, sbkt, scnt)
            bucketize(mtid, mtpid, cnt_t, tbkt, tcnt)

        def ring_wait(_, c):
            pltpu.make_async_copy(souts.at[pl.ds(0, L)],
                                  stag.at[pl.ds(0, L)], sem).wait()
            return c

        def drain(bkt, n, bidx, cb, width, out_hbm, gk):
            def chunk(k, gk):
                packed = bkt[pl.ds(bidx * BCAP + k * L, L)]
                lid16 = lax.shift_right_logical(packed, 14)
                pid16 = packed & (B - 1)
                valid = (lanes + k * L) < n
                col16 = jnp.clip(lid16 - cb, 0, width - 1)
                pid16 = jnp.where(valid, pid16, B)
                lax.fori_loop(0, jnp.where(gk >= 8, 1, 0), ring_wait, 0)
                slot = gk & 7
                ring = slot * L
                for j in range(L):
                    cols = jnp.full((L,), col16[j], jnp.int32)
                    for q in range(D // L):
                        v = plsc.load_gather(win, [lanes + q * L, cols])
                        stag[ring + j, pl.ds(q * L, L)] = v
                pidr[slot] = pid16
                pltpu.async_copy(stag.at[pl.ds(ring, L)],
                                 out_hbm.at[pidr.at[slot]], sem)
                return gk + 1

            return lax.fori_loop(0, (n + L - 1) // L, chunk, gk)

        def window_body(i, gk):
            b0 = pl.multiple_of(base_id + i * WIN, BLK)
            pltpu.sync_copy(embt.at[:, pl.ds(b0, WIN)],
                            win.at[:, pl.ds(0, WIN)])
            gk = drain(sbkt, jnp.minimum(scnt[i], BCAP), i, i * WIN,
                       WIN, souts, gk)
            gk = drain(tbkt, jnp.minimum(tcnt[i], BCAP), i, i * WIN,
                       WIN, touts, gk)
            return gk

        with jax.named_scope("ph_windows"):
            gk = lax.fori_loop(0, NWIN, window_body, jnp.int32(0))

        @pl.when(is_last)
        def _tail():
            pltpu.sync_copy(embt.at[:, pl.ds(TAIL0, TAILN - BLK // 2)],
                            win.at[:, pl.ds(0, TAILN - BLK // 2)])
            pltpu.sync_copy(
                embt.at[:, pl.ds(TAIL0 + TAILN - BLK // 2, BLK // 2)],
                win.at[:, pl.ds(TAILN - BLK // 2, BLK // 2)])
            g2 = gk
            for tw in (NWIN, NWIN + 1):
                g2 = drain(sbkt, jnp.minimum(scnt[tw], BCAP), tw,
                           NWIN * WIN, TAILN, souts, g2)
                g2 = drain(tbkt, jnp.minimum(tcnt[tw], BCAP), tw,
                           NWIN * WIN, TAILN, touts, g2)
            lax.fori_loop(0, jnp.minimum(g2, 8), ring_wait, 0)

        @pl.when(jnp.logical_not(is_last))
        def _final_drain():
            lax.fori_loop(0, jnp.minimum(gk, 8), ring_wait, 0)

    return gather_kernel


_gather_call = _make_gather_kernel()


def _loss_body(s_ref, t_ref, td_ref, cf_ref, out_ref):
    d = (s_ref[pl.ds(0, B), pl.ds(0, D)] -
         t_ref[pl.ds(0, B), pl.ds(0, D)])
    ssq = jnp.sum(d * d, axis=1)
    err = jnp.sqrt(ssq) * 0.125 - td_ref[...]
    out_ref[0, 0] = jnp.sum(err * err * cf_ref[...]) * (1.0 / B)


_loss_call = pl.pallas_call(
    _loss_body,
    out_shape=jax.ShapeDtypeStruct((1, 1), jnp.float32),
    out_specs=pl.BlockSpec(memory_space=pltpu.SMEM),
)


def kernel(embeddings, source_id, target_id, target_distance, confidence):
    sid = source_id.astype(jnp.int32)
    tid = target_id.astype(jnp.int32)
    srows, trows = _gather_call(embeddings.T, sid, tid)
    loss = _loss_call(srows, trows, target_distance, confidence)
    return loss[0, 0]


# speedup vs baseline: 1.0250x; 1.0058x over previous
import functools

import jax
import jax.numpy as jnp
from jax import lax
from jax.experimental import pallas as pl
from jax.experimental.pallas import tpu as pltpu
from jax.experimental.pallas import tpu_sc as plsc

N_EMB = 1000000
D = 64
B = 16384

NC, NS, L = 2, 16, 16
NW = NC * NS
BLK = 128
BPWK = 244
WB = 4
WIN = WB * BLK
NWIN = BPWK // WB
TAIL0 = NW * BPWK * BLK
TAILN = N_EMB - TAIL0
MCAP = 4096
NBKT = 64
BCAP = 64


def _make_gather_kernel():
    mesh = plsc.VectorSubcoreMesh(
        core_axis_name="c", subcore_axis_name="s",
        num_cores=NC, num_subcores=NS)

    @functools.partial(
        pl.kernel,
        out_type=(jax.ShapeDtypeStruct((B + 8, 2 * D), jnp.float32),
                  jax.ShapeDtypeStruct((B + 8, 2 * D), jnp.float32)),
        mesh=mesh,
        scratch_types=[
            pltpu.VMEM((B,), jnp.int32),
            pltpu.VMEM((B,), jnp.int32),
            pltpu.VMEM((MCAP,), jnp.int32),
            pltpu.VMEM((MCAP,), jnp.int32),
            pltpu.VMEM((MCAP,), jnp.int32),
            pltpu.VMEM((MCAP,), jnp.int32),
            pltpu.VMEM((NBKT * BCAP + L,), jnp.int32),
            pltpu.VMEM((NBKT * BCAP + L,), jnp.int32),
            pltpu.SMEM((NBKT,), jnp.int32),
            pltpu.SMEM((NBKT,), jnp.int32),
            pltpu.VMEM((D, TAILN), jnp.float32),
            pltpu.VMEM((8 * L, 2 * D), jnp.float32),
            pltpu.VMEM((8, L), jnp.int32),
            pltpu.VMEM((L,), jnp.int32),
            pltpu.SemaphoreType.DMA,
        ],
        compiler_params=pltpu.CompilerParams(
            needs_layout_passes=False, use_tc_tiling_on_sc=True),
    )
    def gather_kernel(embt, sid, tid, souts, touts, sidv, tidv,
                      msid, mspid, mtid, mtpid, sbkt, tbkt, scnt, tcnt,
                      win, stag, pidr, colb, sem):
        w = lax.axis_index("s") * NC + lax.axis_index("c")
        lo_blk = w * BPWK
        base_id = lo_blk * BLK
        is_last = w == NW - 1
        hi_blk = lo_blk + BPWK + jnp.where(is_last, 5, 0)

        pltpu.sync_copy(sid, sidv)
        pltpu.sync_copy(tid, tidv)
        for b in range(NBKT):
            scnt[b] = 0
            tcnt[b] = 0

        lanes = lax.iota(jnp.int32, L)

        def scan_body(i, offs):
            off_s, off_t = offs
            pidx = lanes + i * L
            s16 = sidv[pl.ds(i * L, L)]
            t16 = tidv[pl.ds(i * L, L)]
            sblk = lax.shift_right_logical(s16, 7)
            tblk = lax.shift_right_logical(t16, 7)
            ms = (sblk >= lo_blk) & (sblk < hi_blk)
            mt = (tblk >= lo_blk) & (tblk < hi_blk)
            plsc.store_compressed(msid.at[pl.ds(off_s, L)], s16, mask=ms)
            plsc.store_compressed(mspid.at[pl.ds(off_s, L)], pidx, mask=ms)
            plsc.store_compressed(mtid.at[pl.ds(off_t, L)], t16, mask=mt)
            plsc.store_compressed(mtpid.at[pl.ds(off_t, L)], pidx, mask=mt)
            ns = plsc.all_reduce_population_count(ms)[0]
            nt = plsc.all_reduce_population_count(mt)[0]
            off_s = jnp.minimum(off_s + ns, MCAP - L)
            off_t = jnp.minimum(off_t + nt, MCAP - L)
            return (off_s, off_t)

        with jax.named_scope("ph_scan"):
            cnt_s, cnt_t = lax.fori_loop(
                0, B // L, scan_body, (jnp.int32(0), jnp.int32(0)))

        def bucketize(mid, mpid, cnt, bkt, bcnt):
            def bk_body(k, carry):
                ids16 = mid[pl.ds(k * L, L)]
                pid16 = mpid[pl.ds(k * L, L)]
                valid = (lanes + k * L) < cnt
                lid16 = ids16 - base_id
                win16 = lax.shift_right_logical(lid16, 9)
                win16 = jnp.where(valid, win16, NBKT - 1)
                packed = (lid16 << 14) | pid16
                for j in range(L):
                    wv = win16[j]
                    slot = bcnt[wv]
                    bcnt[wv] = slot + 1
                    pos = wv * BCAP + jnp.minimum(slot, BCAP - 1)
                    plsc.store_compressed(
                        bkt.at[pl.ds(pos, L)], packed, mask=lanes == j)
                return carry

            lax.fori_loop(0, (cnt + L - 1) // L, bk_body, 0)

        with jax.named_scope("ph_bucket"):
            bucketize(msid, mspid, cnt_s, sbkt, scnt)
            bucketize(mtid, mtpid, cnt_t, tbkt, tcnt)

        def ring_wait(_, c):
            pltpu.make_async_copy(souts.at[pl.ds(0, L)],
                                  stag.at[pl.ds(0, L)], sem).wait()
            return c

        def drain(bkt, n, bidx, cb, width, out_hbm, gk):
            def chunk(k, gk):
                packed = bkt[pl.ds(bidx * BCAP + k * L, L)]
                lid16 = lax.shift_right_logical(packed, 14)
                pid16 = packed & (B - 1)
                valid = (lanes + k * L) < n
                col16 = jnp.clip(lid16 - cb, 0, width - 1)
                pid16 = jnp.where(valid, pid16, B)
                lax.fori_loop(0, jnp.where(gk >= 8, 1, 0), ring_wait, 0)
                slot = gk & 7
                ring = slot * L
                colb[...] = col16
                pidr[slot] = pid16

                def lane(j, c):
                    cols = plsc.load_gather(
                        colb, [jnp.full((L,), j, jnp.int32)])
                    for q in range(D // L):
                        v = plsc.load_gather(win, [lanes + q * L, cols])
                        stag[ring + j, pl.ds(q * L, L)] = v
                    return c

                lax.fori_loop(0, L, lane, 0)
                pltpu.async_copy(stag.at[pl.ds(ring, L)],
                                 out_hbm.at[pidr.at[slot]], sem)
                return gk + 1

            return lax.fori_loop(0, (n + L - 1) // L, chunk, gk)

        def window_body(i, gk):
            b0 = pl.multiple_of(base_id + i * WIN, BLK)
            pltpu.sync_copy(embt.at[:, pl.ds(b0, WIN)],
                            win.at[:, pl.ds(0, WIN)])
            gk = drain(sbkt, jnp.minimum(scnt[i], BCAP), i, i * WIN,
                       WIN, souts, gk)
            gk = drain(tbkt, jnp.minimum(tcnt[i], BCAP), i, i * WIN,
                       WIN, touts, gk)
            return gk

        with jax.named_scope("ph_windows"):
            gk = lax.fori_loop(0, NWIN, window_body, jnp.int32(0))

        @pl.when(is_last)
        def _tail():
            pltpu.sync_copy(embt.at[:, pl.ds(TAIL0, TAILN - BLK // 2)],
                            win.at[:, pl.ds(0, TAILN - BLK // 2)])
            pltpu.sync_copy(
                embt.at[:, pl.ds(TAIL0 + TAILN - BLK // 2, BLK // 2)],
                win.at[:, pl.ds(TAILN - BLK // 2, BLK // 2)])
            g2 = gk
            for tw in (NWIN, NWIN + 1):
                g2 = drain(sbkt, jnp.minimum(scnt[tw], BCAP), tw,
                           NWIN * WIN, TAILN, souts, g2)
                g2 = drain(tbkt, jnp.minimum(tcnt[tw], BCAP), tw,
                           NWIN * WIN, TAILN, touts, g2)
            lax.fori_loop(0, jnp.minimum(g2, 8), ring_wait, 0)

        @pl.when(jnp.logical_not(is_last))
        def _final_drain():
            lax.fori_loop(0, jnp.minimum(gk, 8), ring_wait, 0)

    return gather_kernel


_gather_call = _make_gather_kernel()


def _loss_body(s_ref, t_ref, td_ref, cf_ref, out_ref):
    d = (s_ref[pl.ds(0, B), pl.ds(0, D)] -
         t_ref[pl.ds(0, B), pl.ds(0, D)])
    ssq = jnp.sum(d * d, axis=1)
    err = jnp.sqrt(ssq) * 0.125 - td_ref[...]
    out_ref[0, 0] = jnp.sum(err * err * cf_ref[...]) * (1.0 / B)


_loss_call = pl.pallas_call(
    _loss_body,
    out_shape=jax.ShapeDtypeStruct((1, 1), jnp.float32),
    out_specs=pl.BlockSpec(memory_space=pltpu.SMEM),
)


def kernel(embeddings, source_id, target_id, target_distance, confidence):
    sid = source_id.astype(jnp.int32)
    tid = target_id.astype(jnp.int32)
    srows, trows = _gather_call(embeddings.T, sid, tid)
    loss = _loss_call(srows, trows, target_distance, confidence)
    return loss[0, 0]
